# baseline (device time: 47775 ns/iter reference)
import jax
import jax.numpy as jnp
from jax import lax
from jax.experimental import pallas as pl
from jax.experimental.pallas import tpu as pltpu

N_DEV = 16
NS = 2


def kernel(x, W1, W2):
    m, _ = x.shape
    out_n = W2.shape[1]
    rows = m // N_DEV
    cols = out_n // NS

    def body(x_ref, w1_ref, w2_ref, out_ref, partial2, acc2,
             rs_buf, ag_buf, rs_send, rs_recv, ag_send, ag_recv):
        my_i = lax.axis_index("i")

        xb = x_ref[...].astype(jnp.bfloat16)
        w1 = w1_ref[...].astype(jnp.bfloat16)
        h = jnp.dot(xb, w1, preferred_element_type=jnp.float32)
        hb = jnp.maximum(h, 0.0).astype(jnp.bfloat16)

        rs = [[None] * (N_DEV - 1) for _ in range(NS)]
        for s in range(NS):
            w2s = w2_ref[:, s * cols:(s + 1) * cols].astype(jnp.bfloat16)
            ps = jnp.dot(hb, w2s, preferred_element_type=jnp.float32)
            partial2[s] = ps.astype(jnp.bfloat16)
            for d in range(1, N_DEV):
                tgt = (my_i + d) % N_DEV
                r = pltpu.make_async_remote_copy(
                    src_ref=partial2.at[s, pl.ds(tgt * rows, rows)],
                    dst_ref=rs_buf.at[s, d - 1],
                    send_sem=rs_send.at[s, d - 1],
                    recv_sem=rs_recv.at[s, d - 1],
                    device_id=(tgt,),
                    device_id_type=pl.DeviceIdType.MESH,
                )
                r.start()
                rs[s][d - 1] = r

        ag = [[None] * (N_DEV - 1) for _ in range(NS)]
        for s in range(NS):
            for d in range(1, N_DEV):
                rs[s][d - 1].wait_recv()
            terms = [partial2[s, pl.ds(my_i * rows, rows), :]
                     .astype(jnp.float32)]
            terms += [rs_buf[s, d].astype(jnp.float32)
                      for d in range(N_DEV - 1)]
            while len(terms) > 1:
                terms = [a + b for a, b in zip(terms[::2], terms[1::2])] + (
                    [terms[-1]] if len(terms) % 2 else [])
            acc2[s] = terms[0].astype(jnp.bfloat16)
            for d in range(1, N_DEV):
                tgt = (my_i + d) % N_DEV
                r = pltpu.make_async_remote_copy(
                    src_ref=acc2.at[s],
                    dst_ref=ag_buf.at[s, d - 1],
                    send_sem=ag_send.at[s, d - 1],
                    recv_sem=ag_recv.at[s, d - 1],
                    device_id=(tgt,),
                    device_id_type=pl.DeviceIdType.MESH,
                )
                r.start()
                ag[s][d - 1] = r
            out_ref[pl.ds(my_i * rows, rows), pl.ds(s * cols, cols)] = acc2[s]

        for s in range(NS):
            for d in range(1, N_DEV):
                ag[s][d - 1].wait_recv()
                src = (my_i - d) % N_DEV
                out_ref[pl.ds(src * rows, rows), pl.ds(s * cols, cols)] = (
                    ag_buf[s, d - 1])

        for s in range(NS):
            for d in range(1, N_DEV):
                rs[s][d - 1].wait_send()
                ag[s][d - 1].wait_send()

    return pl.pallas_call(
        body,
        out_shape=jax.ShapeDtypeStruct((m, out_n), jnp.bfloat16),
        in_specs=[pl.BlockSpec(memory_space=pltpu.VMEM)] * 3,
        out_specs=pl.BlockSpec(memory_space=pltpu.VMEM),
        scratch_shapes=[
            pltpu.VMEM((NS, m, cols), jnp.bfloat16),
            pltpu.VMEM((NS, rows, cols), jnp.bfloat16),
            pltpu.VMEM((NS, N_DEV - 1, rows, cols), jnp.bfloat16),
            pltpu.VMEM((NS, N_DEV - 1, rows, cols), jnp.bfloat16),
            pltpu.SemaphoreType.DMA((NS, N_DEV - 1)),
            pltpu.SemaphoreType.DMA((NS, N_DEV - 1)),
            pltpu.SemaphoreType.DMA((NS, N_DEV - 1)),
            pltpu.SemaphoreType.DMA((NS, N_DEV - 1)),
        ],
    )(x, W1, W2)


# device time: 42015 ns/iter; 1.1371x vs baseline; 1.1371x over previous
import jax
import jax.numpy as jnp
from jax import lax
from jax.experimental import pallas as pl
from jax.experimental.pallas import tpu as pltpu

N_DEV = 16


def kernel(x, W1, W2):
    m, _ = x.shape
    out_n = W2.shape[1]
    rows = m // N_DEV

    def body(x_ref, w1_ref, w2_ref, out_ref, partial_ref, acc_ref, rs_buf,
             rs_send, rs_recv, ag_send, ag_recv):
        my_i = lax.axis_index("i")

        xb = x_ref[...].astype(jnp.bfloat16)
        w1 = w1_ref[...].astype(jnp.bfloat16)
        h = jnp.dot(xb, w1, preferred_element_type=jnp.float32)
        hb = jnp.maximum(h, 0.0).astype(jnp.bfloat16)
        w2 = w2_ref[...].astype(jnp.bfloat16)
        partial_f32 = jnp.dot(hb, w2, preferred_element_type=jnp.float32)
        partial_ref[...] = partial_f32.astype(jnp.bfloat16)

        barrier_sem = pltpu.get_barrier_semaphore()
        for d in range(1, N_DEV):
            pl.semaphore_signal(
                barrier_sem, inc=1,
                device_id=((my_i + d) % N_DEV,),
                device_id_type=pl.DeviceIdType.MESH,
            )
        pl.semaphore_wait(barrier_sem, N_DEV - 1)

        rs = []
        for d in range(1, N_DEV):
            tgt = (my_i + d) % N_DEV
            r = pltpu.make_async_remote_copy(
                src_ref=partial_ref.at[pl.ds(tgt * rows, rows)],
                dst_ref=rs_buf.at[d - 1],
                send_sem=rs_send.at[d - 1],
                recv_sem=rs_recv.at[d - 1],
                device_id=(tgt,),
                device_id_type=pl.DeviceIdType.MESH,
            )
            r.start()
            rs.append(r)

        acc = partial_ref[pl.ds(my_i * rows, rows), :].astype(jnp.float32)
        for d in range(1, N_DEV):
            rs[d - 1].wait_recv()
            acc = acc + rs_buf[d - 1].astype(jnp.float32)
        acc_ref[...] = acc.astype(jnp.bfloat16)

        ag = []
        for d in range(1, N_DEV):
            tgt = (my_i + d) % N_DEV
            r = pltpu.make_async_remote_copy(
                src_ref=acc_ref,
                dst_ref=out_ref.at[pl.ds(my_i * rows, rows)],
                send_sem=ag_send.at[d - 1],
                recv_sem=ag_recv.at[d - 1],
                device_id=(tgt,),
                device_id_type=pl.DeviceIdType.MESH,
            )
            r.start()
            ag.append(r)

        out_ref[pl.ds(my_i * rows, rows), :] = acc_ref[...]

        for d in range(1, N_DEV):
            ag[d - 1].wait_recv()
            rs[d - 1].wait_send()
            ag[d - 1].wait_send()

    return pl.pallas_call(
        body,
        out_shape=jax.ShapeDtypeStruct((m, out_n), jnp.bfloat16),
        in_specs=[pl.BlockSpec(memory_space=pltpu.VMEM)] * 3,
        out_specs=pl.BlockSpec(memory_space=pltpu.VMEM),
        compiler_params=pltpu.CompilerParams(collective_id=0),
        scratch_shapes=[
            pltpu.VMEM((m, out_n), jnp.bfloat16),
            pltpu.VMEM((rows, out_n), jnp.bfloat16),
            pltpu.VMEM((N_DEV - 1, rows, out_n), jnp.bfloat16),
            pltpu.SemaphoreType.DMA((N_DEV - 1,)),
            pltpu.SemaphoreType.DMA((N_DEV - 1,)),
            pltpu.SemaphoreType.DMA((N_DEV - 1,)),
            pltpu.SemaphoreType.DMA((N_DEV - 1,)),
        ],
    )(x, W1, W2)
